# baseline (speedup 1.0000x reference)
import functools

import numpy as np

import jax
import jax.numpy as jnp
from jax import lax
from jax.experimental import pallas as pl
from jax.experimental.pallas import tpu as pltpu
from jax.experimental.pallas import tpu_sc as plsc

NNODE = 10000
NEDGE = 320000
DIM = 128

NC = 2
NS = 16
NW = NC * NS
LANES = 16

NPAD = 10240
KCH = 128
CPT_A = 80
CPT_B = 80
TOT_CH = NS * (CPT_A + CPT_B)
EPAD = TOT_CH * KCH
EPT_DEG = EPAD // NW
RPT = NPAD // NS

_PAD_E = np.tile(NNODE + (np.arange(EPAD - NEDGE) % (NPAD - NNODE)),
                 (2, 1)).astype(np.int32)
_ZEROS_ND = np.zeros((NPAD, 1, DIM), np.float32)

_mesh = plsc.VectorSubcoreMesh(
    core_axis_name="c", subcore_axis_name="s", num_cores=NC, num_subcores=NS)


@functools.partial(
    pl.kernel,
    out_type=jax.ShapeDtypeStruct((NW, NPAD), jnp.float32),
    mesh=_mesh,
    scratch_types=[
        pltpu.VMEM((EPT_DEG // KCH, 1, KCH), jnp.int32),
        pltpu.VMEM((NPAD,), jnp.float32),
    ],
    compiler_params=pltpu.CompilerParams(needs_layout_passes=False),
)
def _deg_kernel(ei_hbm, out_hbm, dst_v, deg_v):
    cid = lax.axis_index("c")
    sid = lax.axis_index("s")
    wid = sid * NC + cid
    cpt = EPT_DEG // KCH
    pltpu.sync_copy(ei_hbm.at[1, pl.ds(wid * cpt, cpt)], dst_v)

    @pl.loop(0, NPAD // LANES)
    def _zero(i):
        deg_v[pl.ds(i * LANES, LANES)] = jnp.zeros((LANES,), jnp.float32)

    ones = jnp.ones((LANES,), jnp.float32)

    @pl.loop(0, cpt)
    def _acc(r):
        for k in range(KCH // LANES):
            idx = dst_v[r, 0, pl.ds(k * LANES, LANES)]
            plsc.addupdate_scatter(deg_v, [idx], ones)

    pltpu.sync_copy(deg_v, out_hbm.at[wid])


@functools.partial(
    pl.kernel,
    out_type=jax.ShapeDtypeStruct((NPAD, NC, 1, DIM), jnp.float32),
    mesh=_mesh,
    scratch_types=[
        pltpu.VMEM((CPT_A, 1, KCH), jnp.int32),
        pltpu.VMEM((4, 1, KCH), jnp.int32),
        pltpu.VMEM((2, KCH, 1, DIM), jnp.float32),
        pltpu.VMEM_SHARED((NPAD, 1, DIM), jnp.float32),
        pltpu.SemaphoreType.DMA,
        pltpu.SemaphoreType.DMA,
        pltpu.SemaphoreType.DMA,
        pltpu.SemaphoreType.DMA,
        pltpu.SemaphoreType.DMA,
        pltpu.SemaphoreType.DMA,
    ],
    compiler_params=pltpu.CompilerParams(needs_layout_passes=False),
)
def _spmm_kernel(g_hbm, ei_hbm, zero_hbm, out_hbm,
                 src_v, dst_v, rows_v, acc_sh, sg0, sg1, sd0, sd1, sd2, sd3):
    cid = lax.axis_index("c")
    sid = lax.axis_index("s")
    row0 = sid * RPT
    sg = (sg0, sg1)
    sd = (sd0, sd1, sd2, sd3)

    def edge_pipeline(cbase, cpt):
        pltpu.sync_copy(ei_hbm.at[0, pl.ds(cbase, cpt)],
                        src_v.at[pl.ds(0, cpt)])
        for c in range(4):
            pltpu.async_copy(ei_hbm.at[1, cbase + c], dst_v.at[c], sd[c])
        pltpu.async_copy(g_hbm.at[src_v.at[0, 0]], rows_v.at[0], sg[0])
        pltpu.async_copy(g_hbm.at[src_v.at[1, 0]], rows_v.at[1], sg[1])
        plsc.subcore_barrier()

        @pl.loop(0, cpt, step=4)
        def _quad(j):
            for b in range(4):
                cur = j + b
                rb = b % 2
                pltpu.make_async_copy(g_hbm.at[src_v.at[0, 0]], rows_v.at[rb],
                                      sg[rb]).wait()
                pltpu.make_async_copy(ei_hbm.at[1, cbase], dst_v.at[b],
                                      sd[b]).wait()
                pltpu.sync_copy(rows_v.at[rb], acc_sh.at[dst_v.at[b, 0]],
                                add=True)

                @pl.when(cur + 2 < cpt)
                def _():
                    pltpu.async_copy(g_hbm.at[src_v.at[cur + 2, 0]],
                                     rows_v.at[rb], sg[rb])

                @pl.when(cur + 4 < cpt)
                def _():
                    pltpu.async_copy(ei_hbm.at[1, cbase + cur + 4],
                                     dst_v.at[b], sd[b])

    @pl.when(cid == 0)
    def _():
        pltpu.sync_copy(g_hbm.at[pl.ds(row0, RPT)], acc_sh.at[pl.ds(row0, RPT)])
        edge_pipeline(sid * CPT_A, CPT_A)

    @pl.when(cid != 0)
    def _():
        pltpu.sync_copy(zero_hbm.at[pl.ds(row0, RPT)],
                        acc_sh.at[pl.ds(row0, RPT)])
        edge_pipeline(NS * CPT_A + sid * CPT_B, CPT_B)

    plsc.subcore_barrier()
    pltpu.sync_copy(acc_sh.at[pl.ds(row0, RPT)],
                    out_hbm.at[pl.ds(row0, RPT), cid])


_RB = 1024


def _mm1_body(x_ref, w_ref, degp_ref, g_ref, dinv_ref):
    deg = jnp.sum(degp_ref[...], axis=0) + 1.0
    dinv = lax.rsqrt(deg)
    h = jnp.dot(x_ref[...], w_ref[...], preferred_element_type=jnp.float32)
    g_ref[...] = (h * dinv[:, None])[:, None, :]
    dinv_ref[...] = dinv[:, None]


def _mid_body(t_ref, dinv_ref, w_ref, b_ref, g_ref):
    t = t_ref[:, 0, 0, :] + t_ref[:, 1, 0, :]
    dinv = dinv_ref[...]
    y = t * dinv + b_ref[...]
    h = jnp.dot(y, w_ref[...], preferred_element_type=jnp.float32)
    g_ref[...] = (h * dinv)[:, None, :]


def _fin_body(t_ref, dinv_ref, b_ref, o_ref):
    t = t_ref[:, 0, 0, :] + t_ref[:, 1, 0, :]
    o_ref[...] = t * dinv_ref[...] + b_ref[...]


_GRID = NPAD // _RB

_mm1 = pl.pallas_call(
    _mm1_body,
    grid=(_GRID,),
    in_specs=[
        pl.BlockSpec((_RB, DIM), lambda i: (i, 0)),
        pl.BlockSpec((DIM, DIM), lambda i: (0, 0)),
        pl.BlockSpec((NW, _RB), lambda i: (0, i)),
    ],
    out_specs=[
        pl.BlockSpec((_RB, 1, DIM), lambda i: (i, 0, 0)),
        pl.BlockSpec((_RB, 1), lambda i: (i, 0)),
    ],
    out_shape=[
        jax.ShapeDtypeStruct((NPAD, 1, DIM), jnp.float32),
        jax.ShapeDtypeStruct((NPAD, 1), jnp.float32),
    ],
)

_mid = pl.pallas_call(
    _mid_body,
    grid=(_GRID,),
    in_specs=[
        pl.BlockSpec((_RB, NC, 1, DIM), lambda i: (i, 0, 0, 0)),
        pl.BlockSpec((_RB, 1), lambda i: (i, 0)),
        pl.BlockSpec((DIM, DIM), lambda i: (0, 0)),
        pl.BlockSpec((1, DIM), lambda i: (0, 0)),
    ],
    out_specs=pl.BlockSpec((_RB, 1, DIM), lambda i: (i, 0, 0)),
    out_shape=jax.ShapeDtypeStruct((NPAD, 1, DIM), jnp.float32),
)

_FB = 400

_fin = pl.pallas_call(
    _fin_body,
    grid=(NNODE // _FB,),
    in_specs=[
        pl.BlockSpec((_FB, NC, 1, DIM), lambda i: (i, 0, 0, 0)),
        pl.BlockSpec((_FB, 1), lambda i: (i, 0)),
        pl.BlockSpec((1, DIM), lambda i: (0, 0)),
    ],
    out_specs=pl.BlockSpec((_FB, DIM), lambda i: (i, 0)),
    out_shape=jax.ShapeDtypeStruct((NNODE, DIM), jnp.float32),
)


def kernel(node_features, edge_index, W1, b1, W2, b2):
    xpad = jnp.zeros((NPAD, DIM), jnp.float32).at[:NNODE].set(node_features)
    ei = jnp.concatenate([edge_index, _PAD_E], axis=1)
    ei4 = ei.reshape(2, TOT_CH, 1, KCH)

    degp = _deg_kernel(ei4)
    g1, dinv = _mm1(xpad, W1, degp)
    t1 = _spmm_kernel(g1, ei4, _ZEROS_ND)
    g2 = _mid(t1, dinv, W2, b1.reshape(1, DIM))
    t2 = _spmm_kernel(g2, ei4, _ZEROS_ND)
    return _fin(t2, dinv, b2.reshape(1, DIM))


# speedup vs baseline: 1.0419x; 1.0419x over previous
import functools

import numpy as np

import jax
import jax.numpy as jnp
from jax import lax
from jax.experimental import pallas as pl
from jax.experimental.pallas import tpu as pltpu
from jax.experimental.pallas import tpu_sc as plsc

NNODE = 10000
NEDGE = 320000
DIM = 128

NC = 2
NS = 16
NW = NC * NS
LANES = 16

NPAD = 10240
KCH = 128
CPT_A = 80
CPT_B = 80
TOT_CH = NS * (CPT_A + CPT_B)
EPAD = TOT_CH * KCH
EPT_DEG = EPAD // NW
RPT = NPAD // NS

_PAD_E = np.tile(NNODE + (np.arange(EPAD - NEDGE) % (NPAD - NNODE)),
                 (2, 1)).astype(np.int32)
_ZEROS_ND = np.zeros((NPAD, DIM), np.float32)

_mesh = plsc.VectorSubcoreMesh(
    core_axis_name="c", subcore_axis_name="s", num_cores=NC, num_subcores=NS)


@functools.partial(
    pl.kernel,
    out_type=jax.ShapeDtypeStruct((NW, NPAD), jnp.float32),
    mesh=_mesh,
    scratch_types=[
        pltpu.VMEM((EPT_DEG // KCH, 1, KCH), jnp.int32),
        pltpu.VMEM((NPAD,), jnp.float32),
    ],
    compiler_params=pltpu.CompilerParams(needs_layout_passes=False),
)
def _deg_kernel(ei_hbm, out_hbm, dst_v, deg_v):
    cid = lax.axis_index("c")
    sid = lax.axis_index("s")
    wid = sid * NC + cid
    cpt = EPT_DEG // KCH
    pltpu.sync_copy(ei_hbm.at[1, pl.ds(wid * cpt, cpt)], dst_v)

    @pl.loop(0, NPAD // LANES)
    def _zero(i):
        deg_v[pl.ds(i * LANES, LANES)] = jnp.zeros((LANES,), jnp.float32)

    ones = jnp.ones((LANES,), jnp.float32)

    @pl.loop(0, cpt)
    def _acc(r):
        for k in range(KCH // LANES):
            idx = dst_v[r, 0, pl.ds(k * LANES, LANES)]
            plsc.addupdate_scatter(deg_v, [idx], ones)

    pltpu.sync_copy(deg_v, out_hbm.at[wid])


@functools.partial(
    pl.kernel,
    out_type=jax.ShapeDtypeStruct((NC, NPAD, DIM), jnp.float32),
    mesh=_mesh,
    scratch_types=[
        pltpu.VMEM((CPT_A, 1, KCH), jnp.int32),
        pltpu.VMEM((4, 1, KCH), jnp.int32),
        pltpu.VMEM((2, KCH, DIM), jnp.float32),
        pltpu.VMEM_SHARED((NPAD, DIM), jnp.float32),
        pltpu.SemaphoreType.DMA,
        pltpu.SemaphoreType.DMA,
        pltpu.SemaphoreType.DMA,
        pltpu.SemaphoreType.DMA,
        pltpu.SemaphoreType.DMA,
        pltpu.SemaphoreType.DMA,
    ],
    compiler_params=pltpu.CompilerParams(needs_layout_passes=False),
)
def _spmm_kernel(g_hbm, ei_hbm, zero_hbm, out_hbm,
                 src_v, dst_v, rows_v, acc_sh, sg0, sg1, sd0, sd1, sd2, sd3):
    cid = lax.axis_index("c")
    sid = lax.axis_index("s")
    row0 = sid * RPT
    sg = (sg0, sg1)
    sd = (sd0, sd1, sd2, sd3)

    def edge_pipeline(cbase, cpt):
        pltpu.sync_copy(ei_hbm.at[0, pl.ds(cbase, cpt)],
                        src_v.at[pl.ds(0, cpt)])
        for c in range(4):
            pltpu.async_copy(ei_hbm.at[1, cbase + c], dst_v.at[c], sd[c])
        pltpu.async_copy(g_hbm.at[src_v.at[0, 0]], rows_v.at[0], sg[0])
        pltpu.async_copy(g_hbm.at[src_v.at[1, 0]], rows_v.at[1], sg[1])
        plsc.subcore_barrier()

        @pl.loop(0, cpt, step=4)
        def _quad(j):
            for b in range(4):
                cur = j + b
                rb = b % 2
                pltpu.make_async_copy(g_hbm.at[src_v.at[0, 0]], rows_v.at[rb],
                                      sg[rb]).wait()
                pltpu.make_async_copy(ei_hbm.at[1, cbase], dst_v.at[b],
                                      sd[b]).wait()
                pltpu.sync_copy(rows_v.at[rb], acc_sh.at[dst_v.at[b, 0]],
                                add=True)

                @pl.when(cur + 2 < cpt)
                def _():
                    pltpu.async_copy(g_hbm.at[src_v.at[cur + 2, 0]],
                                     rows_v.at[rb], sg[rb])

                @pl.when(cur + 4 < cpt)
                def _():
                    pltpu.async_copy(ei_hbm.at[1, cbase + cur + 4],
                                     dst_v.at[b], sd[b])

    @pl.when(cid == 0)
    def _():
        pltpu.sync_copy(g_hbm.at[pl.ds(row0, RPT)], acc_sh.at[pl.ds(row0, RPT)])
        edge_pipeline(sid * CPT_A, CPT_A)

    @pl.when(cid != 0)
    def _():
        pltpu.sync_copy(zero_hbm.at[pl.ds(row0, RPT)],
                        acc_sh.at[pl.ds(row0, RPT)])
        edge_pipeline(NS * CPT_A + sid * CPT_B, CPT_B)

    plsc.subcore_barrier()
    pltpu.sync_copy(acc_sh.at[pl.ds(row0, RPT)],
                    out_hbm.at[cid, pl.ds(row0, RPT)])


_RB = 1024


def _mm1_body(x_ref, w_ref, degp_ref, g_ref, dinv_ref):
    deg = jnp.sum(degp_ref[...], axis=0) + 1.0
    dinv = lax.rsqrt(deg)
    h = jnp.dot(x_ref[...], w_ref[...], preferred_element_type=jnp.float32)
    g_ref[...] = h * dinv[:, None]
    dinv_ref[...] = dinv[:, None]


def _mid_body(t_ref, dinv_ref, w_ref, b_ref, g_ref):
    t = t_ref[0] + t_ref[1]
    dinv = dinv_ref[...]
    y = t * dinv + b_ref[...]
    h = jnp.dot(y, w_ref[...], preferred_element_type=jnp.float32)
    g_ref[...] = h * dinv


def _fin_body(t_ref, dinv_ref, b_ref, o_ref):
    t = t_ref[0] + t_ref[1]
    o_ref[...] = t * dinv_ref[...] + b_ref[...]


_GRID = NPAD // _RB

_mm1 = pl.pallas_call(
    _mm1_body,
    grid=(_GRID,),
    in_specs=[
        pl.BlockSpec((_RB, DIM), lambda i: (i, 0)),
        pl.BlockSpec((DIM, DIM), lambda i: (0, 0)),
        pl.BlockSpec((NW, _RB), lambda i: (0, i)),
    ],
    out_specs=[
        pl.BlockSpec((_RB, DIM), lambda i: (i, 0)),
        pl.BlockSpec((_RB, 1), lambda i: (i, 0)),
    ],
    out_shape=[
        jax.ShapeDtypeStruct((NPAD, DIM), jnp.float32),
        jax.ShapeDtypeStruct((NPAD, 1), jnp.float32),
    ],
)

_mid = pl.pallas_call(
    _mid_body,
    grid=(_GRID,),
    in_specs=[
        pl.BlockSpec((NC, _RB, DIM), lambda i: (0, i, 0)),
        pl.BlockSpec((_RB, 1), lambda i: (i, 0)),
        pl.BlockSpec((DIM, DIM), lambda i: (0, 0)),
        pl.BlockSpec((1, DIM), lambda i: (0, 0)),
    ],
    out_specs=pl.BlockSpec((_RB, DIM), lambda i: (i, 0)),
    out_shape=jax.ShapeDtypeStruct((NPAD, DIM), jnp.float32),
)

_FB = 400

_fin = pl.pallas_call(
    _fin_body,
    grid=(NNODE // _FB,),
    in_specs=[
        pl.BlockSpec((NC, _FB, DIM), lambda i: (0, i, 0)),
        pl.BlockSpec((_FB, 1), lambda i: (i, 0)),
        pl.BlockSpec((1, DIM), lambda i: (0, 0)),
    ],
    out_specs=pl.BlockSpec((_FB, DIM), lambda i: (i, 0)),
    out_shape=jax.ShapeDtypeStruct((NNODE, DIM), jnp.float32),
)


def kernel(node_features, edge_index, W1, b1, W2, b2):
    xpad = jnp.zeros((NPAD, DIM), jnp.float32).at[:NNODE].set(node_features)
    ei = jnp.concatenate([edge_index, _PAD_E], axis=1)
    ei4 = ei.reshape(2, TOT_CH, 1, KCH)

    degp = _deg_kernel(ei4)
    g1, dinv = _mm1(xpad, W1, degp)
    t1 = _spmm_kernel(g1, ei4, _ZEROS_ND)
    g2 = _mid(t1, dinv, W2, b1.reshape(1, DIM))
    t2 = _spmm_kernel(g2, ei4, _ZEROS_ND)
    return _fin(t2, dinv, b2.reshape(1, DIM))
